# baseline (device time: 226479 ns/iter reference)
import jax
import jax.numpy as jnp
from jax import lax
from jax.experimental import pallas as pl
from jax.experimental.pallas import tpu as pltpu

N_DEV = 16


def kernel(x, w_mat, scale_x, scale_w):
    m_per, k = x.shape
    _, n_per = w_mat.shape
    sx = scale_x.reshape(1, 1)
    sw = scale_w.reshape(1, 1)

    def body(x_ref, w_ref, sx_ref, sw_ref, out_ref, gather_ref,
             send_sems, recv_sems):
        my = lax.axis_index("i")
        left = lax.rem(my + N_DEV - 1, N_DEV)
        right = lax.rem(my + 1, N_DEV)

        barrier_sem = pltpu.get_barrier_semaphore()
        for nbr in (left, right):
            pl.semaphore_signal(
                barrier_sem, inc=1,
                device_id=(nbr,), device_id_type=pl.DeviceIdType.MESH,
            )
        pl.semaphore_wait(barrier_sem, 2)

        scale = sx_ref[0, 0] * sw_ref[0, 0]

        def compute_chunk(origin, chunk):
            acc = jnp.dot(chunk, w_ref[...], preferred_element_type=jnp.int32)
            y = jnp.maximum(acc.astype(jnp.float32) * scale, 0.0)
            out_ref[pl.ds(origin * m_per, m_per), :] = y

        gather_ref[pl.ds(my * m_per, m_per), :] = x_ref[...]

        for h in range(N_DEV - 1):
            send_origin = lax.rem(my + N_DEV - h, N_DEV)
            recv_origin = lax.rem(my + N_DEV - 1 - h, N_DEV)
            rdma = pltpu.make_async_remote_copy(
                src_ref=gather_ref.at[pl.ds(send_origin * m_per, m_per)],
                dst_ref=gather_ref.at[pl.ds(send_origin * m_per, m_per)],
                send_sem=send_sems.at[h],
                recv_sem=recv_sems.at[h],
                device_id=(right,),
                device_id_type=pl.DeviceIdType.MESH,
            )
            rdma.start()
            if h == 0:
                compute_chunk(my, x_ref[...])
            rdma.wait()
            compute_chunk(
                recv_origin,
                gather_ref[pl.ds(recv_origin * m_per, m_per), :],
            )

    out_shape = jax.ShapeDtypeStruct((N_DEV * m_per, n_per), jnp.float32)
    return pl.pallas_call(
        body,
        out_shape=out_shape,
        in_specs=[
            pl.BlockSpec(memory_space=pltpu.VMEM),
            pl.BlockSpec(memory_space=pltpu.VMEM),
            pl.BlockSpec(memory_space=pltpu.SMEM),
            pl.BlockSpec(memory_space=pltpu.SMEM),
        ],
        out_specs=pl.BlockSpec(memory_space=pltpu.VMEM),
        scratch_shapes=[
            pltpu.VMEM((N_DEV * m_per, k), jnp.int8),
            pltpu.SemaphoreType.DMA((N_DEV - 1,)),
            pltpu.SemaphoreType.DMA((N_DEV - 1,)),
        ],
        compiler_params=pltpu.CompilerParams(collective_id=0),
    )(x, w_mat, sx, sw)


# device time: 100591 ns/iter; 2.2515x vs baseline; 2.2515x over previous
import jax
import jax.numpy as jnp
from jax import lax
from jax.experimental import pallas as pl
from jax.experimental.pallas import tpu as pltpu

N_DEV = 16
S = 2


def kernel(x, w_mat, scale_x, scale_w):
    m_per, k = x.shape
    _, n_per = w_mat.shape
    half = m_per // 2
    piece = half // S
    sx = scale_x.reshape(1, 1)
    sw = scale_w.reshape(1, 1)

    def body(x_ref, w_ref, sx_ref, sw_ref, out_ref, gather_ref,
             cw_send, cw_recv, ccw_send, ccw_recv):
        my = lax.axis_index("i")
        left = lax.rem(my + N_DEV - 1, N_DEV)
        right = lax.rem(my + 1, N_DEV)

        barrier_sem = pltpu.get_barrier_semaphore()
        for nbr in (left, right):
            pl.semaphore_signal(
                barrier_sem, inc=1,
                device_id=(nbr,), device_id_type=pl.DeviceIdType.MESH,
            )
        pl.semaphore_wait(barrier_sem, 2)

        scale = sx_ref[0, 0] * sw_ref[0, 0]

        def cw_rdma(h, p, origin):
            off = origin * m_per + p * piece
            return pltpu.make_async_remote_copy(
                src_ref=gather_ref.at[pl.ds(off, piece)],
                dst_ref=gather_ref.at[pl.ds(off, piece)],
                send_sem=cw_send.at[h, p],
                recv_sem=cw_recv.at[h, p],
                device_id=(right,),
                device_id_type=pl.DeviceIdType.MESH,
            )

        def ccw_rdma(h, p, origin):
            off = origin * m_per + half + p * piece
            return pltpu.make_async_remote_copy(
                src_ref=gather_ref.at[pl.ds(off, piece)],
                dst_ref=gather_ref.at[pl.ds(off, piece)],
                send_sem=ccw_send.at[h, p],
                recv_sem=ccw_recv.at[h, p],
                device_id=(left,),
                device_id_type=pl.DeviceIdType.MESH,
            )

        def compute_half(origin, which):
            row = origin * m_per + which * half
            chunk = gather_ref[pl.ds(row, half), :]
            acc = jnp.dot(chunk, w_ref[...], preferred_element_type=jnp.int32)
            out_ref[pl.ds(row, half), :] = jnp.maximum(
                acc.astype(jnp.float32) * scale, 0.0)

        gather_ref[pl.ds(my * m_per, m_per), :] = x_ref[...]

        for p in range(S):
            cw_rdma(0, p, my).start()
            ccw_rdma(0, p, my).start()

        acc = jnp.dot(x_ref[...], w_ref[...],
                      preferred_element_type=jnp.int32)
        out_ref[pl.ds(my * m_per, m_per), :] = jnp.maximum(
            acc.astype(jnp.float32) * scale, 0.0)

        for h in range(N_DEV - 1):
            cw_origin = lax.rem(my + N_DEV - 1 - h, N_DEV)
            ccw_origin = lax.rem(my + 1 + h, N_DEV)
            for p in range(S):
                cw_rdma(h, p, cw_origin).wait_recv()
                if h < N_DEV - 2:
                    cw_rdma(h + 1, p, cw_origin).start()
                ccw_rdma(h, p, ccw_origin).wait_recv()
                if h < N_DEV - 2:
                    ccw_rdma(h + 1, p, ccw_origin).start()
            compute_half(cw_origin, 0)
            compute_half(ccw_origin, 1)

        for h in range(N_DEV - 1):
            so_cw = lax.rem(my + N_DEV - h, N_DEV)
            so_ccw = lax.rem(my + h, N_DEV)
            for p in range(S):
                cw_rdma(h, p, so_cw).wait_send()
                ccw_rdma(h, p, so_ccw).wait_send()

    out_shape = jax.ShapeDtypeStruct((N_DEV * m_per, n_per), jnp.float32)
    return pl.pallas_call(
        body,
        out_shape=out_shape,
        in_specs=[
            pl.BlockSpec(memory_space=pltpu.VMEM),
            pl.BlockSpec(memory_space=pltpu.VMEM),
            pl.BlockSpec(memory_space=pltpu.SMEM),
            pl.BlockSpec(memory_space=pltpu.SMEM),
        ],
        out_specs=pl.BlockSpec(memory_space=pltpu.VMEM),
        scratch_shapes=[
            pltpu.VMEM((N_DEV * m_per, k), jnp.int8),
            pltpu.SemaphoreType.DMA((N_DEV - 1, S)),
            pltpu.SemaphoreType.DMA((N_DEV - 1, S)),
            pltpu.SemaphoreType.DMA((N_DEV - 1, S)),
            pltpu.SemaphoreType.DMA((N_DEV - 1, S)),
        ],
        compiler_params=pltpu.CompilerParams(collective_id=0),
    )(x, w_mat, sx, sw)
